# baseline (speedup 1.0000x reference)
import dataclasses
import functools

import jax
import jax.numpy as jnp
from jax import lax
from jax.experimental import pallas as pl
from jax.experimental.pallas import tpu as pltpu
from jax.experimental.pallas import tpu_sc as plsc

PAD = 0
B = 1024
L = 200
LANES = 16
LP = 208
V = 30522
VP = 30528
NC = 2
NS = 16
NW = NC * NS
RPW = B // NW
NCHUNK = LP // LANES

_mesh = plsc.VectorSubcoreMesh(core_axis_name="c", subcore_axis_name="s")

_cp = pltpu.CompilerParams()
if "needs_layout_passes" in pltpu.CompilerParams.__dataclass_fields__:
    _cp = dataclasses.replace(_cp, needs_layout_passes=False)
if "use_tc_tiling_on_sc" in pltpu.CompilerParams.__dataclass_fields__:
    _cp = dataclasses.replace(_cp, use_tc_tiling_on_sc=False)


@functools.partial(
    pl.kernel,
    out_type=jax.ShapeDtypeStruct((B, V), jnp.float32),
    mesh=_mesh,
    scratch_types=[
        pltpu.VMEM((RPW, LP), jnp.int32),
        pltpu.VMEM((VP,), jnp.float32),
        pltpu.VMEM((VP,), jnp.float32),
        pltpu.SemaphoreType.DMA,
        pltpu.SemaphoreType.DMA,
    ],
    compiler_params=_cp,
)
def _bow(ids_hbm, out_hbm, ids_v, hist0, hist1, sem0, sem1):
    wid = lax.axis_index("s") * NC + lax.axis_index("c")
    base = wid * RPW

    pltpu.sync_copy(ids_hbm.at[pl.ds(base, RPW)], ids_v)

    zeros_f = jnp.zeros((LANES,), jnp.float32)
    zeros_i = jnp.zeros((LANES,), jnp.int32)
    bufs = (hist0, hist1)
    sems = (sem0, sem1)

    for hist_v in bufs:
        @pl.loop(0, VP, step=LANES)
        def _(i, hist_v=hist_v):
            hist_v[pl.ds(i, LANES)] = zeros_f

    def add_row(hist_v, r):
        carry = zeros_i
        for c in range(NCHUNK):
            ids16 = ids_v[r, pl.ds(c * LANES, LANES)]
            is_pad = ids16 == PAD
            cum = plsc.cumsum(is_pad.astype(jnp.int32))
            valid = (cum + carry) == 0
            cnt, last = plsc.scan_count(ids16, mask=valid)
            plsc.addupdate_scatter(
                hist_v, [ids16], cnt.astype(jnp.float32), mask=last & valid
            )
            carry = carry + plsc.all_reduce_population_count(is_pad)

    def zero_row(hist_v, r):
        for c in range(NCHUNK):
            ids16 = ids_v[r, pl.ds(c * LANES, LANES)]
            plsc.store_scatter(hist_v, [ids16], zeros_f)

    @pl.loop(0, RPW, step=2)
    def _(i):
        for b in range(2):
            r = i + b
            hist_v, sem = bufs[b], sems[b]

            @pl.when(r >= 2)
            def _(hist_v=hist_v, sem=sem, r=r):
                pltpu.make_async_copy(
                    hist_v.at[pl.ds(0, V)], out_hbm.at[base + r - 2], sem
                ).wait()
                zero_row(hist_v, r - 2)

            add_row(hist_v, r)
            pltpu.async_copy(hist_v.at[pl.ds(0, V)], out_hbm.at[base + r], sem)

    pltpu.make_async_copy(
        hist0.at[pl.ds(0, V)], out_hbm.at[base + RPW - 2], sem0
    ).wait()
    pltpu.make_async_copy(
        hist1.at[pl.ds(0, V)], out_hbm.at[base + RPW - 1], sem1
    ).wait()


def kernel(input_ids):
    ids = jnp.pad(input_ids, ((0, 0), (0, LP - L)))
    out = _bow(ids)
    return jnp.maximum(out, 0.0)


# speedup vs baseline: 1.6098x; 1.6098x over previous
import dataclasses
import functools

import jax
import jax.numpy as jnp
from jax import lax
from jax.experimental import pallas as pl
from jax.experimental.pallas import tpu as pltpu
from jax.experimental.pallas import tpu_sc as plsc

PAD = 0
B = 1024
L = 200
LANES = 16
LP = 256
V = 30522
VMAIN = 30464
W = 15232
TAILW = 128
NC = 2
NS = 16
NW = NC * NS
RPW = B // NW
GR = 8
GROUPS = RPW // GR
NCHUNK = LP // LANES

_mesh = plsc.VectorSubcoreMesh(core_axis_name="c", subcore_axis_name="s")

_cp = pltpu.CompilerParams()
if "needs_layout_passes" in pltpu.CompilerParams.__dataclass_fields__:
    _cp = dataclasses.replace(_cp, needs_layout_passes=False)
if "use_tc_tiling_on_sc" in pltpu.CompilerParams.__dataclass_fields__:
    _cp = dataclasses.replace(_cp, use_tc_tiling_on_sc=True)


@functools.partial(
    pl.kernel,
    out_type=(
        jax.ShapeDtypeStruct((B, V), jnp.float32),
        jax.ShapeDtypeStruct((B, TAILW), jnp.float32),
    ),
    mesh=_mesh,
    scratch_types=[
        pltpu.VMEM((GR, LP), jnp.int32),
        pltpu.VMEM((GR, W), jnp.float32),
        pltpu.VMEM((GR, TAILW), jnp.float32),
    ],
    compiler_params=_cp,
)
def _bow(ids_hbm, out_hbm, tail_hbm, ids_v, buf, tailbuf):
    wid = lax.axis_index("s") * NC + lax.axis_index("c")
    base = wid * RPW

    zeros_f = jnp.zeros((LANES,), jnp.float32)
    zeros_i = jnp.zeros((LANES,), jnp.int32)

    for r in range(GR):
        @pl.loop(0, W, step=LANES)
        def _(i, r=r):
            buf[r, pl.ds(i, LANES)] = zeros_f

        @pl.loop(0, TAILW, step=LANES)
        def _(i, r=r):
            tailbuf[r, pl.ds(i, LANES)] = zeros_f

    @pl.loop(0, GROUPS)
    def _(g):
        rg = base + g * GR
        pltpu.sync_copy(ids_hbm.at[pl.ds(rg, GR)], ids_v)

        for lo in (0, W):
            hi = lo + W

            @pl.loop(0, GR)
            def _(r8, lo=lo, hi=hi):
                r8v = jnp.full((LANES,), r8, jnp.int32)
                carry = zeros_i
                for c in range(NCHUNK):
                    ids16 = ids_v[r8, pl.ds(c * LANES, LANES)]
                    is_pad = ids16 == PAD
                    cum = plsc.cumsum(is_pad.astype(jnp.int32))
                    valid = (cum + carry) == 0
                    carry = carry + plsc.all_reduce_population_count(is_pad)
                    cnt, last = plsc.scan_count(ids16, mask=valid)
                    sel = last & valid
                    cntf = cnt.astype(jnp.float32)
                    m = sel & (ids16 >= lo) & (ids16 < hi)
                    rel = jnp.where(m, ids16 - lo, 0)
                    plsc.addupdate_scatter(buf, [r8v, rel], cntf, mask=m)
                    if hi == VMAIN:
                        mt = sel & (ids16 >= VMAIN)
                        relt = jnp.where(mt, ids16 - VMAIN, 0)
                        plsc.addupdate_scatter(
                            tailbuf, [r8v, relt], cntf, mask=mt
                        )

            pltpu.sync_copy(buf, out_hbm.at[pl.ds(rg, GR), pl.ds(lo, W)])

            @pl.loop(0, GR)
            def _(r8, lo=lo, hi=hi):
                r8v = jnp.full((LANES,), r8, jnp.int32)
                for c in range(NCHUNK):
                    ids16 = ids_v[r8, pl.ds(c * LANES, LANES)]
                    inh = (ids16 >= lo) & (ids16 < hi)
                    rel = jnp.where(inh, ids16 - lo, 0)
                    plsc.store_scatter(buf, [r8v, rel], zeros_f, mask=inh)

        pltpu.sync_copy(tailbuf, tail_hbm.at[pl.ds(rg, GR)])

        @pl.loop(0, GR)
        def _(r8):
            r8v = jnp.full((LANES,), r8, jnp.int32)
            for c in range(NCHUNK):
                ids16 = ids_v[r8, pl.ds(c * LANES, LANES)]
                mt = ids16 >= VMAIN
                relt = jnp.where(mt, ids16 - VMAIN, 0)
                plsc.store_scatter(tailbuf, [r8v, relt], zeros_f, mask=mt)


def kernel(input_ids):
    ids = jnp.pad(input_ids, ((0, 0), (0, LP - L)))
    main, tail = _bow(ids)
    tail58 = lax.slice(tail, (0, 0), (B, V - VMAIN))
    return lax.dynamic_update_slice(main, tail58, (0, VMAIN))
